# baseline (speedup 1.0000x reference)
import functools

import jax
import jax.numpy as jnp
from jax import lax
from jax.experimental import pallas as pl
from jax.experimental.pallas import tpu as pltpu
from jax.experimental.pallas import tpu_sc as plsc

N_EDGES = 320000
N_NODES = 10000
D = 128
LANES = 8

NUM_WORKERS = 32
SC_CHUNK = 200
NCHUNK_SC = 25
SC_ROWS = NUM_WORKERS * SC_CHUNK * NCHUNK_SC
TC_ROWS = N_EDGES - SC_ROWS

REDUCE_BLOCK = 6400
ACC_ROWS = 256
NODE_BLOCK = 2000


def _sc_reduce_body(neigh_hbm, out_hbm, buf0, buf1, accv, sem0, sem1):
    wid = lax.axis_index("s") * 2 + lax.axis_index("c")
    rows_per = SC_CHUNK * NCHUNK_SC
    base = wid * rows_per
    bufs = (buf0, buf1)
    sems = (sem0, sem1)

    pltpu.async_copy(neigh_hbm.at[pl.ds(base, SC_CHUNK)], buf0, sem0)
    if NCHUNK_SC > 1:
        pltpu.async_copy(neigh_hbm.at[pl.ds(base + SC_CHUNK, SC_CHUNK)],
                         buf1, sem1)

    def accum_chunk(buf, accs):
        def blk(b, accs):
            r0 = b * 8
            for rr in range(8):
                accs = tuple(
                    accs[j] + buf[r0 + rr, pl.ds(16 * j, 16)]
                    for j in range(LANES)
                )
            return accs
        return lax.fori_loop(0, SC_CHUNK // 8, blk, accs)

    accs = tuple(jnp.zeros((16,), jnp.float32) for _ in range(LANES))
    for c in range(NCHUNK_SC):
        slot = c % 2
        pltpu.make_async_copy(neigh_hbm.at[pl.ds(base, SC_CHUNK)],
                              bufs[slot], sems[slot]).wait()
        accs = accum_chunk(bufs[slot], accs)
        if c + 2 < NCHUNK_SC:
            pltpu.async_copy(
                neigh_hbm.at[pl.ds(base + (c + 2) * SC_CHUNK, SC_CHUNK)],
                bufs[slot], sems[slot])

    for j in range(LANES):
        accv[0, pl.ds(16 * j, 16)] = accs[j]
    pltpu.sync_copy(accv, out_hbm.at[pl.ds(wid, 1)])


def _sc_partial_sums(neighbor_all):
    mesh = plsc.VectorSubcoreMesh(core_axis_name="c", subcore_axis_name="s")
    k = functools.partial(
        pl.kernel,
        out_type=jax.ShapeDtypeStruct((NUM_WORKERS, D), jnp.float32),
        mesh=mesh,
        scratch_types=[
            pltpu.VMEM((SC_CHUNK, D), jnp.float32),
            pltpu.VMEM((SC_CHUNK, D), jnp.float32),
            pltpu.VMEM((1, D), jnp.float32),
            pltpu.SemaphoreType.DMA,
            pltpu.SemaphoreType.DMA,
        ],
    )(_sc_reduce_body)
    return k(neighbor_all)


def _tc_reduce_body(x_ref, out_ref):
    step = pl.program_id(0)

    @pl.when(step == 0)
    def _():
        out_ref[...] = jnp.zeros_like(out_ref)

    x = x_ref[...]
    partial = jnp.sum(x.reshape(REDUCE_BLOCK // ACC_ROWS, ACC_ROWS, D), axis=0)
    out_ref[...] += partial


def _tc_partial_sums(neighbor_all):
    base_blk = SC_ROWS // REDUCE_BLOCK
    return pl.pallas_call(
        _tc_reduce_body,
        grid=(TC_ROWS // REDUCE_BLOCK,),
        in_specs=[pl.BlockSpec((REDUCE_BLOCK, D), lambda i: (base_blk + i, 0))],
        out_specs=pl.BlockSpec((ACC_ROWS, D), lambda i: (0, 0)),
        out_shape=jax.ShapeDtypeStruct((ACC_ROWS, D), jnp.float32),
    )(neighbor_all)


def _fused_body(src_ref, w_self_ref, w_aggr_ref, sc_sums_ref, tc_sums_ref,
                out_ref):
    self_hidden = jnp.dot(src_ref[...], w_self_ref[...],
                          preferred_element_type=jnp.float32)
    total = jnp.sum(sc_sums_ref[...], axis=0, keepdims=True)
    if tc_sums_ref is not None:
        total = total + jnp.sum(tc_sums_ref[...], axis=0, keepdims=True)
    mean = total * (1.0 / N_EDGES)
    nh = jnp.dot(mean, w_aggr_ref[...], preferred_element_type=jnp.float32)
    out_ref[:, :D] = jnp.maximum(self_hidden, 0.0)
    out_ref[:, D:] = jnp.broadcast_to(jnp.maximum(nh, 0.0),
                                      (out_ref.shape[0], D))


def kernel(src_node_features, neighbor_node_features, W_aggr, W_self):
    if TC_ROWS > 0:
        tc_sums = _tc_partial_sums(neighbor_node_features)
    sc_sums = _sc_partial_sums(neighbor_node_features)
    operands = [src_node_features, W_self, W_aggr, sc_sums]
    in_specs = [
        pl.BlockSpec((NODE_BLOCK, D), lambda i: (i, 0)),
        pl.BlockSpec((D, D), lambda i: (0, 0)),
        pl.BlockSpec((D, D), lambda i: (0, 0)),
        pl.BlockSpec((NUM_WORKERS, D), lambda i: (0, 0)),
    ]
    if TC_ROWS > 0:
        operands.append(tc_sums)
        in_specs.append(pl.BlockSpec((ACC_ROWS, D), lambda i: (0, 0)))
        body = _fused_body
    else:
        def body(src_ref, w_self_ref, w_aggr_ref, sc_sums_ref, out_ref):
            _fused_body(src_ref, w_self_ref, w_aggr_ref, sc_sums_ref,
                        None, out_ref)

    out = pl.pallas_call(
        body,
        grid=(N_NODES // NODE_BLOCK,),
        in_specs=in_specs,
        out_specs=pl.BlockSpec((NODE_BLOCK, 2 * D), lambda i: (i, 0)),
        out_shape=jax.ShapeDtypeStruct((N_NODES, 2 * D), jnp.float32),
    )(*operands)
    return out


# speedup vs baseline: 1.0021x; 1.0021x over previous
import functools

import jax
import jax.numpy as jnp
from jax import lax
from jax.experimental import pallas as pl
from jax.experimental.pallas import tpu as pltpu
from jax.experimental.pallas import tpu_sc as plsc

N_EDGES = 320000
N_NODES = 10000
D = 128
LANES = 8

NUM_WORKERS = 32
SC_CHUNK = 200
NCHUNK_SC = 25
SC_ROWS = NUM_WORKERS * SC_CHUNK * NCHUNK_SC
TC_ROWS = N_EDGES - SC_ROWS

REDUCE_BLOCK = 6400
ACC_ROWS = 256
NODE_BLOCK = 2000


def _sc_reduce_body(neigh_hbm, out_hbm, buf0, buf1, accv, sem0, sem1):
    wid = lax.axis_index("s") * 2 + lax.axis_index("c")
    rows_per = SC_CHUNK * NCHUNK_SC
    base = wid * rows_per
    bufs = (buf0, buf1)
    sems = (sem0, sem1)

    pltpu.async_copy(neigh_hbm.at[pl.ds(base, SC_CHUNK)], buf0, sem0)
    if NCHUNK_SC > 1:
        pltpu.async_copy(neigh_hbm.at[pl.ds(base + SC_CHUNK, SC_CHUNK)],
                         buf1, sem1)

    def accum_chunk(buf, accs):
        def blk(b, accs):
            r0 = b * 8
            for rr in range(8):
                accs = tuple(
                    accs[j] + buf[r0 + rr, pl.ds(16 * j, 16)]
                    for j in range(LANES)
                )
            return accs
        return lax.fori_loop(0, SC_CHUNK // 8, blk, accs)

    accs = tuple(jnp.zeros((16,), jnp.float32) for _ in range(LANES))
    for c in range(NCHUNK_SC):
        slot = c % 2
        pltpu.make_async_copy(neigh_hbm.at[pl.ds(base, SC_CHUNK)],
                              bufs[slot], sems[slot]).wait()
        accs = accum_chunk(bufs[slot], accs)
        if c + 2 < NCHUNK_SC:
            pltpu.async_copy(
                neigh_hbm.at[pl.ds(base + (c + 2) * SC_CHUNK, SC_CHUNK)],
                bufs[slot], sems[slot])

    for j in range(LANES):
        accv[0, pl.ds(16 * j, 16)] = accs[j]
    pltpu.sync_copy(accv, out_hbm.at[pl.ds(wid, 1)])


def _sc_partial_sums(neighbor_all):
    mesh = plsc.VectorSubcoreMesh(core_axis_name="c", subcore_axis_name="s")
    k = functools.partial(
        pl.kernel,
        out_type=jax.ShapeDtypeStruct((NUM_WORKERS, D), jnp.float32),
        mesh=mesh,
        scratch_types=[
            pltpu.VMEM((SC_CHUNK, D), jnp.float32),
            pltpu.VMEM((SC_CHUNK, D), jnp.float32),
            pltpu.VMEM((1, D), jnp.float32),
            pltpu.SemaphoreType.DMA,
            pltpu.SemaphoreType.DMA,
        ],
    )(_sc_reduce_body)
    return k(neighbor_all)


def _tc_reduce_body(x_ref, out_ref):
    step = pl.program_id(0)

    @pl.when(step == 0)
    def _():
        out_ref[...] = jnp.zeros_like(out_ref)

    x = x_ref[...]
    partial = jnp.sum(x.reshape(REDUCE_BLOCK // ACC_ROWS, ACC_ROWS, D), axis=0)
    out_ref[...] += partial


def _tc_partial_sums(neighbor_all):
    base_blk = SC_ROWS // REDUCE_BLOCK
    return pl.pallas_call(
        _tc_reduce_body,
        grid=(TC_ROWS // REDUCE_BLOCK,),
        in_specs=[pl.BlockSpec((REDUCE_BLOCK, D), lambda i: (base_blk + i, 0))],
        out_specs=pl.BlockSpec((ACC_ROWS, D), lambda i: (0, 0)),
        out_shape=jax.ShapeDtypeStruct((ACC_ROWS, D), jnp.float32),
    )(neighbor_all)


def _fused_body(src_ref, w_self_ref, w_aggr_ref, sc_sums_ref, tc_sums_ref,
                out_ref):
    self_hidden = jnp.dot(src_ref[...], w_self_ref[...],
                          preferred_element_type=jnp.float32)
    total = jnp.sum(sc_sums_ref[...], axis=0, keepdims=True)
    if tc_sums_ref is not None:
        total = total + jnp.sum(tc_sums_ref[...], axis=0, keepdims=True)
    mean = total * (1.0 / N_EDGES)
    nh = jnp.dot(mean, w_aggr_ref[...], preferred_element_type=jnp.float32)
    out_ref[:, :D] = jnp.maximum(self_hidden, 0.0)
    out_ref[:, D:] = jnp.broadcast_to(jnp.maximum(nh, 0.0),
                                      (out_ref.shape[0], D))


def kernel(src_node_features, neighbor_node_features, W_aggr, W_self):
    from jax.experimental import scheduling_groups

    @scheduling_groups.scheduling_group("gcn_reduce")
    def _both(neigh):
        return _sc_partial_sums(neigh), _tc_partial_sums(neigh)

    sc_sums, tc_sums = _both(neighbor_node_features)
    operands = [src_node_features, W_self, W_aggr, sc_sums]
    in_specs = [
        pl.BlockSpec((NODE_BLOCK, D), lambda i: (i, 0)),
        pl.BlockSpec((D, D), lambda i: (0, 0)),
        pl.BlockSpec((D, D), lambda i: (0, 0)),
        pl.BlockSpec((NUM_WORKERS, D), lambda i: (0, 0)),
    ]
    if TC_ROWS > 0:
        operands.append(tc_sums)
        in_specs.append(pl.BlockSpec((ACC_ROWS, D), lambda i: (0, 0)))
        body = _fused_body
    else:
        def body(src_ref, w_self_ref, w_aggr_ref, sc_sums_ref, out_ref):
            _fused_body(src_ref, w_self_ref, w_aggr_ref, sc_sums_ref,
                        None, out_ref)

    out = pl.pallas_call(
        body,
        grid=(N_NODES // NODE_BLOCK,),
        in_specs=in_specs,
        out_specs=pl.BlockSpec((NODE_BLOCK, 2 * D), lambda i: (i, 0)),
        out_shape=jax.ShapeDtypeStruct((N_NODES, 2 * D), jnp.float32),
    )(*operands)
    return out
